# Initial kernel scaffold; baseline (speedup 1.0000x reference)
import jax
import jax.numpy as jnp
from jax.experimental import pallas as pl


def kernel(inputs, W1, Wg, Wp, Wb, pat_index):
    raise NotImplementedError("write your pallas kernel here")


# speedup vs baseline: 1.0543x; 1.0543x over previous
import jax
import jax.numpy as jnp
from jax.experimental import pallas as pl
from jax.experimental.pallas import tpu as pltpu
from functools import partial

N_TOK = 4096
D_MODEL = 1024
D_FF = 2048
N_OUT = 3

ROWS = 512


def _body(x_ref, w1_ref, wg_ref, wcat_ref, o_ref):
    x = x_ref[...]
    g = jax.lax.dot_general(
        x, wg_ref[...], (((1,), (0,)), ((), ())),
        preferred_element_type=jnp.float32)
    h = jnp.maximum(
        jax.lax.dot_general(
            x, w1_ref[...], (((1,), (0,)), ((), ())),
            preferred_element_type=jnp.float32),
        0.0)
    pb = jax.lax.dot_general(
        h, wcat_ref[...], (((1,), (0,)), ((), ())),
        preferred_element_type=jnp.float32)
    o_ref[...] = jnp.where(g > 0.0, pb[:, :N_OUT], pb[:, N_OUT:])


@jax.jit
def kernel(inputs, W1, Wg, Wp, Wb, pat_index):
    S = jnp.zeros((D_FF, N_OUT), dtype=Wp.dtype).at[pat_index].add(Wp)
    Wcat = jnp.concatenate([S, Wb], axis=1)

    grid = (N_TOK // ROWS,)
    out = pl.pallas_call(
        _body,
        grid=grid,
        in_specs=[
            pl.BlockSpec((ROWS, D_MODEL), lambda i: (i, 0)),
            pl.BlockSpec((D_MODEL, D_FF), lambda i: (0, 0)),
            pl.BlockSpec((D_MODEL, 1), lambda i: (0, 0)),
            pl.BlockSpec((D_FF, 2 * N_OUT), lambda i: (0, 0)),
        ],
        out_specs=pl.BlockSpec((ROWS, N_OUT), lambda i: (i, 0)),
        out_shape=jax.ShapeDtypeStruct((N_TOK, N_OUT), inputs.dtype),
        compiler_params=pltpu.CompilerParams(
            dimension_semantics=("arbitrary",),
        ),
    )(inputs, W1, Wg, Wcat)
    return out
